# Initial kernel scaffold; baseline (speedup 1.0000x reference)
import jax
import jax.numpy as jnp
from jax.experimental import pallas as pl


def kernel(x, edge_index, edge_attr, u, batch, W1, b1, gamma, beta, W2, b2):
    raise NotImplementedError("write your pallas kernel here")


# speedup vs baseline: 2.3304x; 2.3304x over previous
import functools

import jax
import jax.numpy as jnp
from jax import lax
from jax.experimental import pallas as pl
from jax.experimental.pallas import tpu as pltpu
from jax.experimental.pallas import tpu_sc as plsc

N = 10000
G = 128
H = 256
OUT = 2
EPS = 1e-5

L = 16
NC = 2
NS = 16
NW = NC * NS
RW = 320
GRP = 16


def _seg_sum_sc(x, batch):
    mesh = plsc.VectorSubcoreMesh(core_axis_name="c", subcore_axis_name="s")

    @functools.partial(
        pl.kernel,
        mesh=mesh,
        compiler_params=pltpu.CompilerParams(needs_layout_passes=False),
        out_type=[
            jax.ShapeDtypeStruct((NW, G, H), jnp.float32),
            jax.ShapeDtypeStruct((NW, G, 128), jnp.float32),
        ],
        scratch_types=[
            pltpu.VMEM((GRP,), jnp.int32),
            pltpu.VMEM((GRP, H), jnp.float32),
            pltpu.VMEM((G, H), jnp.float32),
            pltpu.VMEM((G, 128), jnp.float32),
        ],
    )
    def k(x_hbm, b_hbm, out_hbm, cnt_hbm, bidx, xt, acc, cnt):
        cid = lax.axis_index("c")
        sid = lax.axis_index("s")
        wid = sid * NC + cid
        base = wid * RW
        ngrp = jnp.clip(N - base, 0, RW) // GRP

        zeros16 = jnp.zeros((L,), jnp.float32)

        def zrow(i, carry):
            for c in range(H // L):
                acc[i, pl.ds(c * L, L)] = zeros16
            return carry

        def zcnt(i, carry):
            cnt[i, pl.ds(0, L)] = zeros16
            return carry

        lax.fori_loop(0, G, zrow, 0)
        lax.fori_loop(0, G, zcnt, 0)

        col_iota = lax.iota(jnp.int32, L)
        ones16 = jnp.ones((L,), jnp.float32)

        def body(g, carry):
            r0 = base + g * GRP
            pltpu.sync_copy(b_hbm.at[pl.ds(r0, GRP)], bidx)
            pltpu.sync_copy(x_hbm.at[pl.ds(r0, GRP)], xt)
            b16 = bidx[...]
            for j in range(GRP):
                seg = b16[j]
                segv = jnp.full((L,), seg, jnp.int32)
                plsc.addupdate_scatter(cnt, [segv, col_iota], ones16)
                for c in range(H // L):
                    xv = xt[j, pl.ds(c * L, L)]
                    plsc.addupdate_scatter(acc, [segv, col_iota + c * L], xv)
            return carry

        lax.fori_loop(0, ngrp, body, 0)
        pltpu.sync_copy(acc, out_hbm.at[wid])
        pltpu.sync_copy(cnt, cnt_hbm.at[wid])

    return k(x, batch)


def _mlp_body(p_ref, c_ref, w1_ref, b1_ref, g_ref, be_ref, w2_ref, b2_ref, o_ref):
    seg = jnp.sum(p_ref[...], axis=0)
    cnt = jnp.sum(c_ref[...], axis=0)[:, 0:1]
    mean = seg / jnp.maximum(cnt, 1.0)
    h = jnp.dot(mean, w1_ref[...], preferred_element_type=jnp.float32) + b1_ref[...]
    mu = jnp.mean(h, axis=0, keepdims=True)
    var = jnp.mean((h - mu) ** 2, axis=0, keepdims=True)
    hn = (h - mu) * lax.rsqrt(var + EPS) * g_ref[...] + be_ref[...]
    hr = jnp.maximum(hn, 0.0)
    o_ref[...] = jnp.dot(hr, w2_ref[...], preferred_element_type=jnp.float32) + b2_ref[...]


def kernel(x, edge_index, edge_attr, u, batch, W1, b1, gamma, beta, W2, b2):
    del edge_index, edge_attr, u
    batch_i32 = batch.astype(jnp.int32)
    partials, cnts = _seg_sum_sc(x, batch_i32)

    w2p = jnp.zeros((H, 128), jnp.float32).at[:, :OUT].set(W2)
    b2p = jnp.zeros((1, 128), jnp.float32).at[:, :OUT].set(b2[None, :])

    out = pl.pallas_call(
        _mlp_body,
        out_shape=jax.ShapeDtypeStruct((G, 128), jnp.float32),
    )(
        partials,
        cnts,
        W1,
        b1[None, :],
        gamma[None, :],
        beta[None, :],
        w2p,
        b2p,
    )
    return out[:, :OUT]


# speedup vs baseline: 3.3445x; 1.4352x over previous
import functools

import jax
import jax.numpy as jnp
from jax import lax
from jax.experimental import pallas as pl
from jax.experimental.pallas import tpu as pltpu
from jax.experimental.pallas import tpu_sc as plsc

N = 10000
G = 128
H = 256
OUT = 2
EPS = 1e-5

L = 16
NC = 2
NS = 16
NW = NC * NS
RW = 320
CH = 80
GRP = 16
NPAD = 10240


def _seg_sum_sc(x, batch):
    mesh = plsc.VectorSubcoreMesh(core_axis_name="c", subcore_axis_name="s")

    @functools.partial(
        pl.kernel,
        mesh=mesh,
        compiler_params=pltpu.CompilerParams(needs_layout_passes=False),
        out_type=jax.ShapeDtypeStruct((NW, G, H), jnp.float32),
        scratch_types=[
            pltpu.VMEM((RW,), jnp.int32),
            pltpu.VMEM((RW, H), jnp.float32),
            pltpu.VMEM((G, H), jnp.float32),
            pltpu.SemaphoreType.DMA,
        ],
    )
    def k(x_hbm, b_hbm, out_hbm, bidx, xt, acc, sem):
        cid = lax.axis_index("c")
        sid = lax.axis_index("s")
        wid = sid * NC + cid
        base = wid * RW
        rows = jnp.clip(N - base, 0, RW)

        for c in range(RW // CH):
            @pl.when(c * CH < rows)
            def _():
                pltpu.async_copy(
                    b_hbm.at[pl.ds(base + c * CH, CH)],
                    bidx.at[pl.ds(c * CH, CH)],
                    sem,
                )
                pltpu.async_copy(
                    x_hbm.at[pl.ds(base + c * CH, CH)],
                    xt.at[pl.ds(c * CH, CH)],
                    sem,
                )

        zeros16 = jnp.zeros((L,), jnp.float32)

        def zrow(i, carry):
            for c in range(H // L):
                acc[i, pl.ds(c * L, L)] = zeros16
            return carry

        lax.fori_loop(0, G, zrow, 0)

        for c in range(RW // CH):
            @pl.when(c * CH < rows)
            def _():
                pltpu.make_async_copy(
                    b_hbm.at[pl.ds(base + c * CH, CH)],
                    bidx.at[pl.ds(c * CH, CH)],
                    sem,
                ).wait()
                pltpu.make_async_copy(
                    x_hbm.at[pl.ds(base + c * CH, CH)],
                    xt.at[pl.ds(c * CH, CH)],
                    sem,
                ).wait()

        col_iota = lax.iota(jnp.int32, L)

        def body(g, carry):
            b16 = bidx[pl.ds(g * GRP, GRP)]
            for j in range(GRP):
                seg = b16[j]
                segv = jnp.full((L,), seg, jnp.int32)
                for c in range(H // L):
                    xv = xt[g * GRP + j, pl.ds(c * L, L)]
                    plsc.addupdate_scatter(acc, [segv, col_iota + c * L], xv)
            return carry

        lax.fori_loop(0, rows // GRP, body, 0)
        pltpu.sync_copy(acc, out_hbm.at[wid])

    return k(x, batch)


def _mlp_body(p_ref, b_ref, w1_ref, b1_ref, g_ref, be_ref, w2_ref, b2_ref, o_ref):
    seg = jnp.sum(p_ref[...], axis=0)
    ids = lax.broadcasted_iota(jnp.int32, (1, 1, G), 2)
    cnt = jnp.sum(
        (b_ref[...][:, :, None] == ids).astype(jnp.float32), axis=(0, 1)
    )[:, None]
    mean = seg / jnp.maximum(cnt, 1.0)
    h = jnp.dot(mean, w1_ref[...], preferred_element_type=jnp.float32) + b1_ref[...]
    mu = jnp.mean(h, axis=0, keepdims=True)
    var = jnp.mean((h - mu) ** 2, axis=0, keepdims=True)
    hn = (h - mu) * lax.rsqrt(var + EPS) * g_ref[...] + be_ref[...]
    hr = jnp.maximum(hn, 0.0)
    o_ref[...] = jnp.dot(hr, w2_ref[...], preferred_element_type=jnp.float32) + b2_ref[...]


def kernel(x, edge_index, edge_attr, u, batch, W1, b1, gamma, beta, W2, b2):
    del edge_index, edge_attr, u
    batch_i32 = batch.astype(jnp.int32)
    partials = _seg_sum_sc(x, batch_i32)

    bpad = jnp.concatenate(
        [batch_i32, jnp.full((NPAD - N,), G, jnp.int32)]
    ).reshape(NPAD // G, G)
    w2p = jnp.zeros((H, 128), jnp.float32).at[:, :OUT].set(W2)
    b2p = jnp.zeros((1, 128), jnp.float32).at[:, :OUT].set(b2[None, :])

    out = pl.pallas_call(
        _mlp_body,
        out_shape=jax.ShapeDtypeStruct((G, 128), jnp.float32),
    )(
        partials,
        bpad,
        W1,
        b1[None, :],
        gamma[None, :],
        beta[None, :],
        w2p,
        b2p,
    )
    return out[:, :OUT]


# speedup vs baseline: 3.9901x; 1.1930x over previous
import functools

import jax
import jax.numpy as jnp
from jax import lax
from jax.experimental import pallas as pl
from jax.experimental.pallas import tpu as pltpu
from jax.experimental.pallas import tpu_sc as plsc

N = 10000
G = 128
H = 256
OUT = 2
EPS = 1e-5

L = 16
NC = 2
NS = 16
NW = NC * NS
RW = 320
CH = 80
GRP = 16
NPAD = 10240


def _tree_sum(vs):
    vs = list(vs)
    while len(vs) > 1:
        vs = [a + b for a, b in zip(vs[::2], vs[1::2])]
    return vs[0]


def _seg_sum_sc(x, batch):
    mesh = plsc.VectorSubcoreMesh(core_axis_name="c", subcore_axis_name="s")

    @functools.partial(
        pl.kernel,
        mesh=mesh,
        compiler_params=pltpu.CompilerParams(needs_layout_passes=False),
        out_type=jax.ShapeDtypeStruct((NW, G, H), jnp.float32),
        scratch_types=[
            pltpu.VMEM((RW,), jnp.int32),
            pltpu.VMEM((RW, H), jnp.float32),
            pltpu.VMEM((G, H), jnp.float32),
            pltpu.SemaphoreType.DMA,
        ],
    )
    def k(x_hbm, b_hbm, out_hbm, bidx, xt, acc, sem):
        cid = lax.axis_index("c")
        sid = lax.axis_index("s")
        wid = sid * NC + cid
        base = wid * RW
        rows = jnp.clip(N - base, 0, RW)

        for c in range(RW // CH):
            @pl.when(c * CH < rows)
            def _():
                pltpu.async_copy(
                    b_hbm.at[pl.ds(base + c * CH, CH)],
                    bidx.at[pl.ds(c * CH, CH)],
                    sem,
                )
                pltpu.async_copy(
                    x_hbm.at[pl.ds(base + c * CH, CH)],
                    xt.at[pl.ds(c * CH, CH)],
                    sem,
                )

        zeros16 = jnp.zeros((L,), jnp.float32)

        def zrow(i, carry):
            for c in range(H // L):
                acc[i, pl.ds(c * L, L)] = zeros16
            return carry

        lax.fori_loop(0, G, zrow, 0)

        for c in range(RW // CH):
            @pl.when(c * CH < rows)
            def _():
                pltpu.make_async_copy(
                    b_hbm.at[pl.ds(base + c * CH, CH)],
                    bidx.at[pl.ds(c * CH, CH)],
                    sem,
                ).wait()
                pltpu.make_async_copy(
                    x_hbm.at[pl.ds(base + c * CH, CH)],
                    xt.at[pl.ds(c * CH, CH)],
                    sem,
                ).wait()

        col_iota = lax.iota(jnp.int32, L)

        def body(g, carry):
            r0 = g * GRP
            b16 = bidx[pl.ds(r0, GRP)]
            seg0 = b16[0]
            segv0 = jnp.full((L,), seg0, jnp.int32)
            uniform = jnp.all(b16 == segv0)

            @pl.when(uniform)
            def _():
                for c in range(H // L):
                    s = _tree_sum(
                        [xt[r0 + j, pl.ds(c * L, L)] for j in range(GRP)]
                    )
                    plsc.addupdate_scatter(acc, [segv0, col_iota + c * L], s)

            @pl.when(jnp.logical_not(uniform))
            def _():
                for j in range(GRP):
                    seg = b16[j]
                    segv = jnp.full((L,), seg, jnp.int32)
                    for c in range(H // L):
                        xv = xt[r0 + j, pl.ds(c * L, L)]
                        plsc.addupdate_scatter(
                            acc, [segv, col_iota + c * L], xv
                        )

            return carry

        lax.fori_loop(0, rows // GRP, body, 0)
        pltpu.sync_copy(acc, out_hbm.at[wid])

    return k(x, batch)


def _mlp_body(p_ref, b_ref, w1_ref, b1_ref, g_ref, be_ref, w2_ref, b2_ref, o_ref):
    seg = jnp.sum(p_ref[...], axis=0)
    ids = lax.broadcasted_iota(jnp.int32, (1, 1, G), 2)
    cnt = jnp.sum(
        (b_ref[...][:, :, None] == ids).astype(jnp.float32), axis=(0, 1)
    )[:, None]
    mean = seg / jnp.maximum(cnt, 1.0)
    h = jnp.dot(mean, w1_ref[...], preferred_element_type=jnp.float32) + b1_ref[...]
    mu = jnp.mean(h, axis=0, keepdims=True)
    var = jnp.mean((h - mu) ** 2, axis=0, keepdims=True)
    hn = (h - mu) * lax.rsqrt(var + EPS) * g_ref[...] + be_ref[...]
    hr = jnp.maximum(hn, 0.0)
    o_ref[...] = jnp.dot(hr, w2_ref[...], preferred_element_type=jnp.float32) + b2_ref[...]


def kernel(x, edge_index, edge_attr, u, batch, W1, b1, gamma, beta, W2, b2):
    del edge_index, edge_attr, u
    batch_i32 = batch.astype(jnp.int32)
    partials = _seg_sum_sc(x, batch_i32)

    bpad = jnp.concatenate(
        [batch_i32, jnp.full((NPAD - N,), G, jnp.int32)]
    ).reshape(NPAD // G, G)
    w2p = jnp.zeros((H, 128), jnp.float32).at[:, :OUT].set(W2)
    b2p = jnp.zeros((1, 128), jnp.float32).at[:, :OUT].set(b2[None, :])

    out = pl.pallas_call(
        _mlp_body,
        out_shape=jax.ShapeDtypeStruct((G, 128), jnp.float32),
    )(
        partials,
        bpad,
        W1,
        b1[None, :],
        gamma[None, :],
        beta[None, :],
        w2p,
        b2p,
    )
    return out[:, :OUT]
